# baseline (speedup 1.0000x reference)
import jax
import jax.numpy as jnp
from jax.experimental import pallas as pl
from jax.experimental.pallas import tpu as pltpu

_BLOCK = 768


def _body(scale_ref, w1_ref, w2_ref, nf_ref, nb_ref, out_ref):
    e, f = w1_ref.shape
    z = jnp.zeros((e, f), jnp.float32)
    w = jnp.concatenate(
        [jnp.concatenate([w1_ref[...], z], axis=1),
         jnp.concatenate([z, w2_ref[...]], axis=1)], axis=0)
    nb = nb_ref[...]
    mean = jnp.sum(nb, axis=1) * (1.0 / nb.shape[1])
    x = jnp.concatenate([nf_ref[...], mean], axis=1)
    y = jax.lax.dot_general(
        w, x, (((1,), (1,)), ((), ())),
        preferred_element_type=jnp.float32)
    out_ref[...] = jnp.maximum(y * scale_ref[0], 0.0)


def kernel(node_feats, neighbor_feats, weight_1, weight_2, node_count):
    n, f = node_feats.shape
    s = neighbor_feats.shape[0] // n
    e = weight_1.shape[0]
    scale = (jnp.asarray(node_count, jnp.float32) / jnp.float32(n)).reshape(1)
    nb3 = neighbor_feats.reshape(n, s, f)
    b = _BLOCK
    grid = (n + b - 1) // b
    return pl.pallas_call(
        _body,
        grid=(grid,),
        in_specs=[
            pl.BlockSpec(memory_space=pltpu.SMEM),
            pl.BlockSpec((e, f), lambda i: (0, 0)),
            pl.BlockSpec((e, f), lambda i: (0, 0)),
            pl.BlockSpec((b, f), lambda i: (i, 0)),
            pl.BlockSpec((b, s, f), lambda i: (i, 0, 0)),
        ],
        out_specs=pl.BlockSpec((2 * e, b), lambda i: (0, i)),
        out_shape=jax.ShapeDtypeStruct((2 * e, n), jnp.float32),
    )(scale, weight_1, weight_2, node_feats, nb3)


# speedup vs baseline: 1.0246x; 1.0246x over previous
import jax
import jax.numpy as jnp
from jax.experimental import pallas as pl
from jax.experimental.pallas import tpu as pltpu

_BLOCK = 512


def _body(scale_ref, w1_ref, w2_ref, nf_ref, nb_ref, out_ref):
    e, f = w1_ref.shape
    z = jnp.zeros((e, f), jnp.float32)
    w = jnp.concatenate(
        [jnp.concatenate([w1_ref[...], z], axis=1),
         jnp.concatenate([z, w2_ref[...]], axis=1)], axis=0)
    nb = nb_ref[...]
    mean = jnp.sum(nb, axis=1) * (1.0 / nb.shape[1])
    x = jnp.concatenate([nf_ref[...], mean], axis=1)
    y = jax.lax.dot_general(
        w, x, (((1,), (1,)), ((), ())),
        preferred_element_type=jnp.float32)
    out_ref[...] = jnp.maximum(y * scale_ref[0], 0.0)


def kernel(node_feats, neighbor_feats, weight_1, weight_2, node_count):
    n, f = node_feats.shape
    s = neighbor_feats.shape[0] // n
    e = weight_1.shape[0]
    scale = (jnp.asarray(node_count, jnp.float32) / jnp.float32(n)).reshape(1)
    nb3 = neighbor_feats.reshape(n, s, f)
    b = _BLOCK
    grid = (n + b - 1) // b
    return pl.pallas_call(
        _body,
        grid=(grid,),
        in_specs=[
            pl.BlockSpec(memory_space=pltpu.SMEM),
            pl.BlockSpec((e, f), lambda i: (0, 0)),
            pl.BlockSpec((e, f), lambda i: (0, 0)),
            pl.BlockSpec((b, f), lambda i: (i, 0)),
            pl.BlockSpec((b, s, f), lambda i: (i, 0, 0)),
        ],
        out_specs=pl.BlockSpec((2 * e, b), lambda i: (0, i)),
        out_shape=jax.ShapeDtypeStruct((2 * e, n), jnp.float32),
    )(scale, weight_1, weight_2, node_feats, nb3)
